# baseline (speedup 1.0000x reference)
import functools

import jax
import jax.numpy as jnp
from jax import lax
from jax.experimental import pallas as pl
from jax.experimental.pallas import tpu as pltpu
from jax.experimental.pallas import tpu_sc as plsc

N = 10000
E = 320000
D_HID = 32

NW = 32
CH = 80
NCH = 125
ROWS_PT = 632
N_ACC = 16 * ROWS_PT

_MESH = plsc.VectorSubcoreMesh(core_axis_name="c", subcore_axis_name="s")


def _prop_body(u_hbm, src_hbm, dst_hbm, zeros_hbm, out_hbm,
               sidx_v, didx_v, rows0, rows1, rows2, rows3, acc_sh,
               gsem0, gsem1, gsem2, gsem3, ssem0, ssem1, ssem2, ssem3):
    c = lax.axis_index("c")
    s = lax.axis_index("s")
    wid = c * 16 + s
    r0 = s * ROWS_PT
    pltpu.sync_copy(zeros_hbm.at[pl.ds(r0, ROWS_PT)],
                    acc_sh.at[pl.ds(r0, ROWS_PT)])
    pltpu.sync_copy(src_hbm.at[wid], sidx_v)
    pltpu.sync_copy(dst_hbm.at[wid], didx_v)
    plsc.subcore_barrier()

    rows = [rows0, rows1, rows2, rows3]
    gsem = [gsem0, gsem1, gsem2, gsem3]
    ssem = [ssem0, ssem1, ssem2, ssem3]

    def g_issue(i, b):
        pltpu.async_copy(u_hbm.at[sidx_v.at[i]], rows[b], gsem[b])

    def g_wait(b):
        pltpu.make_async_copy(u_hbm.at[sidx_v.at[0]], rows[b], gsem[b]).wait()

    def s_issue(i, b):
        pltpu.async_copy(rows[b], acc_sh.at[didx_v.at[i]], ssem[b], add=True)

    def s_wait(b):
        pltpu.make_async_copy(u_hbm.at[sidx_v.at[0]], rows[b], ssem[b]).wait()

    g_issue(0, 0)
    g_issue(1, 1)
    g_issue(2, 2)
    g_wait(0)
    s_issue(0, 0)
    g_issue(3, 3)
    g_wait(1)
    s_issue(1, 1)

    def group(g, carry):
        k0 = 4 * g + 2
        for b in range(4):
            k = k0 + b
            bb = (b + 2) % 4
            s_wait(b)
            g_issue(k + 2, b)
            g_wait(bb)
            s_issue(k, bb)
        return carry

    lax.fori_loop(0, (NCH - 5) // 4, group, 0)
    s_wait(0)
    g_issue(NCH - 1, 0)
    g_wait(2)
    s_issue(NCH - 3, 2)
    s_wait(1)
    g_wait(3)
    s_issue(NCH - 2, 3)
    s_wait(2)
    g_wait(0)
    s_issue(NCH - 1, 0)
    s_wait(3)
    s_wait(0)
    plsc.subcore_barrier()
    pltpu.sync_copy(acc_sh.at[pl.ds(r0, ROWS_PT)],
                    out_hbm.at[c, pl.ds(r0, ROWS_PT)])


_SC_PARAMS = pltpu.CompilerParams(use_tc_tiling_on_sc=False)

_prop = pl.kernel(
    _prop_body,
    mesh=_MESH,
    compiler_params=_SC_PARAMS,
    out_type=jax.ShapeDtypeStruct((2, N_ACC, D_HID), jnp.float32),
    scratch_types=[
        pltpu.VMEM((NCH, CH), jnp.int32),
        pltpu.VMEM((NCH, CH), jnp.int32),
        pltpu.VMEM((CH, D_HID), jnp.float32),
        pltpu.VMEM((CH, D_HID), jnp.float32),
        pltpu.VMEM((CH, D_HID), jnp.float32),
        pltpu.VMEM((CH, D_HID), jnp.float32),
        pltpu.VMEM_SHARED((N_ACC, D_HID), jnp.float32),
        pltpu.SemaphoreType.DMA,
        pltpu.SemaphoreType.DMA,
        pltpu.SemaphoreType.DMA,
        pltpu.SemaphoreType.DMA,
        pltpu.SemaphoreType.DMA,
        pltpu.SemaphoreType.DMA,
        pltpu.SemaphoreType.DMA,
        pltpu.SemaphoreType.DMA,
    ],
)


def _deg_body(dst_hbm, ones_hbm, zeros_hbm, out_hbm, didx_v, ones_v, acc_sh,
              ssem):
    c = lax.axis_index("c")
    s = lax.axis_index("s")
    wid = c * 16 + s
    r0 = s * ROWS_PT
    pltpu.sync_copy(zeros_hbm.at[pl.ds(r0, ROWS_PT)],
                    acc_sh.at[pl.ds(r0, ROWS_PT)])
    pltpu.sync_copy(ones_hbm, ones_v)
    pltpu.sync_copy(dst_hbm.at[wid], didx_v)
    plsc.subcore_barrier()

    def fire(i, carry):
        pltpu.async_copy(ones_v, acc_sh.at[didx_v.at[i]], ssem, add=True)
        return carry

    lax.fori_loop(0, NCH, fire, 0)

    def drain(i, carry):
        pltpu.make_async_copy(ones_hbm, ones_v, ssem).wait()
        return carry

    lax.fori_loop(0, NCH, drain, 0)
    plsc.subcore_barrier()
    pltpu.sync_copy(acc_sh.at[pl.ds(r0, ROWS_PT)],
                    out_hbm.at[c, pl.ds(r0, ROWS_PT)])


D_DEG = 16

_deg = pl.kernel(
    _deg_body,
    mesh=_MESH,
    compiler_params=_SC_PARAMS,
    out_type=jax.ShapeDtypeStruct((2, N_ACC, D_DEG), jnp.float32),
    scratch_types=[
        pltpu.VMEM((NCH, CH), jnp.int32),
        pltpu.VMEM((CH, D_DEG), jnp.float32),
        pltpu.VMEM_SHARED((N_ACC, D_DEG), jnp.float32),
        pltpu.SemaphoreType.DMA,
    ],
)


def _tc_b0_body(x_ref, w1_ref, h1_ref):
    h1_ref[...] = jnp.dot(x_ref[...], w1_ref[...],
                          preferred_element_type=jnp.float32)


_tc_b0 = pl.pallas_call(
    _tc_b0_body,
    out_shape=jax.ShapeDtypeStruct((N, D_HID), jnp.float32),
)


def _tc_b1_body(h1_ref, degp_ref, u1_ref, dinv_ref):
    deg = degp_ref[0, :N, 0:1] + degp_ref[1, :N, 0:1] + 1.0
    dinv = lax.rsqrt(deg)
    u1_ref[:N, :] = h1_ref[...] * dinv
    u1_ref[N:, :] = jnp.zeros((N_ACC - N, D_HID), jnp.float32)
    dinv_ref[...] = dinv


_tc_b1 = pl.pallas_call(
    _tc_b1_body,
    out_shape=(
        jax.ShapeDtypeStruct((N_ACC, D_HID), jnp.float32),
        jax.ShapeDtypeStruct((N, 1), jnp.float32),
    ),
)


def _tc_d_body(p_ref, u1_ref, dinv_ref, b1_ref, u2_ref):
    t = p_ref[0, :N, :] + p_ref[1, :N, :] + u1_ref[:N, :]
    h = jnp.maximum(dinv_ref[...] * t + b1_ref[...], 0.0)
    u2_ref[:N, :] = dinv_ref[...] * h
    u2_ref[N:, :] = jnp.zeros((N_ACC - N, D_HID), jnp.float32)


_tc_d = pl.pallas_call(
    _tc_d_body,
    out_shape=jax.ShapeDtypeStruct((N_ACC, D_HID), jnp.float32),
)


def _tc_f_body(p_ref, u2_ref, dinv_ref, w2_ref, b2_ref, out_ref):
    g = dinv_ref[...] * (p_ref[0, :N, :] + p_ref[1, :N, :] + u2_ref[:N, :])
    out_ref[...] = jnp.dot(g, w2_ref[...],
                           preferred_element_type=jnp.float32) + b2_ref[...]


def kernel(x, edge_index, W1, b1, W2, b2):
    out_ch = W2.shape[1]
    tc_f = pl.pallas_call(
        _tc_f_body,
        out_shape=jax.ShapeDtypeStruct((N, out_ch), jnp.float32),
    )

    dst3 = edge_index[1].reshape(NW, NCH, CH)

    zeros32 = jnp.zeros((N_ACC, D_HID), jnp.float32)
    zeros_deg = jnp.zeros((N_ACC, D_DEG), jnp.float32)
    ones = jnp.ones((CH, D_DEG), jnp.float32)

    degp = _deg(dst3, ones, zeros_deg)
    h1 = _tc_b0(x, W1)
    edge_d = lax.optimization_barrier((edge_index, degp))[0]
    src3 = edge_d[0].reshape(NW, NCH, CH)
    u1, dinv = _tc_b1(h1, degp)
    p1 = _prop(u1, src3, dst3, zeros32)
    u2 = _tc_d(p1, u1, dinv, b1.reshape(1, D_HID))
    p2 = _prop(u2, src3, dst3, zeros32)
    out = tc_f(p2, u2, dinv, W2, b2.reshape(1, out_ch))
    return out


# speedup vs baseline: 1.1539x; 1.1539x over previous
import functools

import jax
import jax.numpy as jnp
from jax import lax
from jax.experimental import pallas as pl
from jax.experimental.pallas import tpu as pltpu
from jax.experimental.pallas import tpu_sc as plsc

N = 10000
E = 320000
D_HID = 32

NW = 32
CH = 80
NCH = 125
ROWS_PT = 640
N_ACC = 16 * ROWS_PT
ROWS_PW = N_ACC // NW

_MESH = plsc.VectorSubcoreMesh(core_axis_name="c", subcore_axis_name="s")


def _prop_body(u_hbm, src_hbm, dst_hbm, zeros_hbm, out_hbm,
               sidx_v, didx_v, rows0, rows1, rows2, rows3, acc_sh,
               gsem0, gsem1, gsem2, gsem3, ssem0, ssem1, ssem2, ssem3):
    c = lax.axis_index("c")
    s = lax.axis_index("s")
    wid = c * 16 + s
    r0 = s * ROWS_PT
    pltpu.sync_copy(zeros_hbm.at[pl.ds(r0, ROWS_PT)],
                    acc_sh.at[pl.ds(r0, ROWS_PT)])
    pltpu.sync_copy(src_hbm.at[wid], sidx_v)
    pltpu.sync_copy(dst_hbm.at[wid], didx_v)
    plsc.subcore_barrier()

    rows = [rows0, rows1, rows2, rows3]
    gsem = [gsem0, gsem1, gsem2, gsem3]
    ssem = [ssem0, ssem1, ssem2, ssem3]

    def g_issue(i, b):
        pltpu.async_copy(u_hbm.at[sidx_v.at[i]], rows[b], gsem[b])

    def g_wait(b):
        pltpu.make_async_copy(u_hbm.at[sidx_v.at[0]], rows[b], gsem[b]).wait()

    def s_issue(i, b):
        pltpu.async_copy(rows[b], acc_sh.at[didx_v.at[i]], ssem[b], add=True)

    def s_wait(b):
        pltpu.make_async_copy(u_hbm.at[sidx_v.at[0]], rows[b], ssem[b]).wait()

    g_issue(0, 0)
    g_issue(1, 1)
    g_issue(2, 2)
    g_wait(0)
    s_issue(0, 0)
    g_issue(3, 3)
    g_wait(1)
    s_issue(1, 1)

    def group(g, carry):
        k0 = 4 * g + 2
        for b in range(4):
            k = k0 + b
            bb = (b + 2) % 4
            s_wait(b)
            g_issue(k + 2, b)
            g_wait(bb)
            s_issue(k, bb)
        return carry

    lax.fori_loop(0, (NCH - 5) // 4, group, 0)
    s_wait(0)
    g_issue(NCH - 1, 0)
    g_wait(2)
    s_issue(NCH - 3, 2)
    s_wait(1)
    g_wait(3)
    s_issue(NCH - 2, 3)
    s_wait(2)
    g_wait(0)
    s_issue(NCH - 1, 0)
    s_wait(3)
    s_wait(0)
    plsc.subcore_barrier()
    pltpu.sync_copy(acc_sh.at[pl.ds(r0, ROWS_PT)],
                    out_hbm.at[c, pl.ds(r0, ROWS_PT)])


_SC_PARAMS = pltpu.CompilerParams(use_tc_tiling_on_sc=False)

_prop = pl.kernel(
    _prop_body,
    mesh=_MESH,
    compiler_params=_SC_PARAMS,
    out_type=jax.ShapeDtypeStruct((2, N_ACC, D_HID), jnp.float32),
    scratch_types=[
        pltpu.VMEM((NCH, CH), jnp.int32),
        pltpu.VMEM((NCH, CH), jnp.int32),
        pltpu.VMEM((CH, D_HID), jnp.float32),
        pltpu.VMEM((CH, D_HID), jnp.float32),
        pltpu.VMEM((CH, D_HID), jnp.float32),
        pltpu.VMEM((CH, D_HID), jnp.float32),
        pltpu.VMEM_SHARED((N_ACC, D_HID), jnp.float32),
        pltpu.SemaphoreType.DMA,
        pltpu.SemaphoreType.DMA,
        pltpu.SemaphoreType.DMA,
        pltpu.SemaphoreType.DMA,
        pltpu.SemaphoreType.DMA,
        pltpu.SemaphoreType.DMA,
        pltpu.SemaphoreType.DMA,
        pltpu.SemaphoreType.DMA,
    ],
)


def _deg_body(dst_hbm, ones_hbm, zeros_hbm, out_hbm, didx_v, ones_v, acc_sh,
              ssem):
    c = lax.axis_index("c")
    s = lax.axis_index("s")
    wid = c * 16 + s
    r0 = s * ROWS_PT
    pltpu.sync_copy(zeros_hbm.at[pl.ds(r0, ROWS_PT)],
                    acc_sh.at[pl.ds(r0, ROWS_PT)])
    pltpu.sync_copy(ones_hbm, ones_v)
    pltpu.sync_copy(dst_hbm.at[wid], didx_v)
    plsc.subcore_barrier()

    def fire(i, carry):
        pltpu.async_copy(ones_v, acc_sh.at[didx_v.at[i]], ssem, add=True)
        return carry

    lax.fori_loop(0, NCH, fire, 0)

    def drain(i, carry):
        pltpu.make_async_copy(ones_hbm, ones_v, ssem).wait()
        return carry

    lax.fori_loop(0, NCH, drain, 0)
    plsc.subcore_barrier()
    pltpu.sync_copy(acc_sh.at[pl.ds(r0, ROWS_PT)],
                    out_hbm.at[c, pl.ds(r0, ROWS_PT)])


D_DEG = 16

_deg = pl.kernel(
    _deg_body,
    mesh=_MESH,
    compiler_params=_SC_PARAMS,
    out_type=jax.ShapeDtypeStruct((2, N_ACC, D_DEG), jnp.float32),
    scratch_types=[
        pltpu.VMEM((NCH, CH), jnp.int32),
        pltpu.VMEM((CH, D_DEG), jnp.float32),
        pltpu.VMEM_SHARED((N_ACC, D_DEG), jnp.float32),
        pltpu.SemaphoreType.DMA,
    ],
)


def _tc_b0_body(x_ref, w1_ref, h1_ref):
    h1_ref[:N, :] = jnp.dot(x_ref[...], w1_ref[...],
                            preferred_element_type=jnp.float32)
    h1_ref[N:, :] = jnp.zeros((N_ACC - N, D_HID), jnp.float32)


_tc_b0 = pl.pallas_call(
    _tc_b0_body,
    out_shape=jax.ShapeDtypeStruct((N_ACC, D_HID), jnp.float32),
)


def _rsqrt16(v):
    i = lax.bitcast_convert_type(v, jnp.int32)
    i = jnp.int32(0x5F3759DF) - lax.shift_right_logical(i, 1)
    y = lax.bitcast_convert_type(i, jnp.float32)
    for _ in range(3):
        y = y * (1.5 - 0.5 * v * y * y)
    return y


def _b1_body(degp_hbm, h1_hbm, u1_hbm, dinv_hbm, p0_v, p1_v, h1_v, u1_v, dv_v):
    wid = lax.axis_index("c") * 16 + lax.axis_index("s")
    r0 = wid * ROWS_PW
    pltpu.sync_copy(degp_hbm.at[0, pl.ds(r0, ROWS_PW)], p0_v)
    pltpu.sync_copy(degp_hbm.at[1, pl.ds(r0, ROWS_PW)], p1_v)
    pltpu.sync_copy(h1_hbm.at[pl.ds(r0, ROWS_PW)], h1_v)

    def row(j, carry):
        deg = p0_v[j, :] + p1_v[j, :] + 1.0
        dinv = _rsqrt16(deg)
        dv_v[j, :] = dinv
        u1_v[j, pl.ds(0, 16)] = dinv * h1_v[j, pl.ds(0, 16)]
        u1_v[j, pl.ds(16, 16)] = dinv * h1_v[j, pl.ds(16, 16)]
        return carry

    lax.fori_loop(0, ROWS_PW, row, 0)
    pltpu.sync_copy(u1_v, u1_hbm.at[pl.ds(r0, ROWS_PW)])
    pltpu.sync_copy(dv_v, dinv_hbm.at[pl.ds(r0, ROWS_PW)])


_b1 = pl.kernel(
    _b1_body,
    mesh=_MESH,
    compiler_params=_SC_PARAMS,
    out_type=(
        jax.ShapeDtypeStruct((N_ACC, D_HID), jnp.float32),
        jax.ShapeDtypeStruct((N_ACC, D_DEG), jnp.float32),
    ),
    scratch_types=[
        pltpu.VMEM((ROWS_PW, D_DEG), jnp.float32),
        pltpu.VMEM((ROWS_PW, D_DEG), jnp.float32),
        pltpu.VMEM((ROWS_PW, D_HID), jnp.float32),
        pltpu.VMEM((ROWS_PW, D_HID), jnp.float32),
        pltpu.VMEM((ROWS_PW, D_DEG), jnp.float32),
    ],
)


def _d_body(p_hbm, u1_hbm, dinv_hbm, b1_hbm, u2_hbm,
            p0_v, p1_v, u1_v, dv_v, b1_v, u2_v):
    wid = lax.axis_index("c") * 16 + lax.axis_index("s")
    r0 = wid * ROWS_PW
    pltpu.sync_copy(p_hbm.at[0, pl.ds(r0, ROWS_PW)], p0_v)
    pltpu.sync_copy(p_hbm.at[1, pl.ds(r0, ROWS_PW)], p1_v)
    pltpu.sync_copy(u1_hbm.at[pl.ds(r0, ROWS_PW)], u1_v)
    pltpu.sync_copy(dinv_hbm.at[pl.ds(r0, ROWS_PW)], dv_v)
    pltpu.sync_copy(b1_hbm, b1_v)
    b1_lo = b1_v[0, pl.ds(0, 16)]
    b1_hi = b1_v[0, pl.ds(16, 16)]

    def row(j, carry):
        dv = dv_v[j, :]
        t_lo = p0_v[j, pl.ds(0, 16)] + p1_v[j, pl.ds(0, 16)] + u1_v[j, pl.ds(0, 16)]
        t_hi = p0_v[j, pl.ds(16, 16)] + p1_v[j, pl.ds(16, 16)] + u1_v[j, pl.ds(16, 16)]
        h_lo = jnp.maximum(dv * t_lo + b1_lo, 0.0)
        h_hi = jnp.maximum(dv * t_hi + b1_hi, 0.0)
        u2_v[j, pl.ds(0, 16)] = dv * h_lo
        u2_v[j, pl.ds(16, 16)] = dv * h_hi
        return carry

    lax.fori_loop(0, ROWS_PW, row, 0)
    pltpu.sync_copy(u2_v, u2_hbm.at[pl.ds(r0, ROWS_PW)])


_dml = pl.kernel(
    _d_body,
    mesh=_MESH,
    compiler_params=_SC_PARAMS,
    out_type=jax.ShapeDtypeStruct((N_ACC, D_HID), jnp.float32),
    scratch_types=[
        pltpu.VMEM((ROWS_PW, D_HID), jnp.float32),
        pltpu.VMEM((ROWS_PW, D_HID), jnp.float32),
        pltpu.VMEM((ROWS_PW, D_HID), jnp.float32),
        pltpu.VMEM((ROWS_PW, D_DEG), jnp.float32),
        pltpu.VMEM((1, D_HID), jnp.float32),
        pltpu.VMEM((ROWS_PW, D_HID), jnp.float32),
    ],
)


def _g_body(p_hbm, u2_hbm, dinv_hbm, g_hbm, p0_v, p1_v, u2_v, dv_v, g_v):
    wid = lax.axis_index("c") * 16 + lax.axis_index("s")
    r0 = wid * ROWS_PW
    pltpu.sync_copy(p_hbm.at[0, pl.ds(r0, ROWS_PW)], p0_v)
    pltpu.sync_copy(p_hbm.at[1, pl.ds(r0, ROWS_PW)], p1_v)
    pltpu.sync_copy(u2_hbm.at[pl.ds(r0, ROWS_PW)], u2_v)
    pltpu.sync_copy(dinv_hbm.at[pl.ds(r0, ROWS_PW)], dv_v)

    def row(j, carry):
        dv = dv_v[j, :]
        lo = pl.ds(0, 16)
        hi = pl.ds(16, 16)
        g_v[j, lo] = dv * (p0_v[j, lo] + p1_v[j, lo] + u2_v[j, lo])
        g_v[j, hi] = dv * (p0_v[j, hi] + p1_v[j, hi] + u2_v[j, hi])
        return carry

    lax.fori_loop(0, ROWS_PW, row, 0)
    pltpu.sync_copy(g_v, g_hbm.at[pl.ds(r0, ROWS_PW)])


_g = pl.kernel(
    _g_body,
    mesh=_MESH,
    compiler_params=_SC_PARAMS,
    out_type=jax.ShapeDtypeStruct((N_ACC, D_HID), jnp.float32),
    scratch_types=[
        pltpu.VMEM((ROWS_PW, D_HID), jnp.float32),
        pltpu.VMEM((ROWS_PW, D_HID), jnp.float32),
        pltpu.VMEM((ROWS_PW, D_HID), jnp.float32),
        pltpu.VMEM((ROWS_PW, D_DEG), jnp.float32),
        pltpu.VMEM((ROWS_PW, D_HID), jnp.float32),
    ],
)


def _tc_f_body(g_ref, w2_ref, b2_ref, out_ref):
    out_ref[...] = jnp.dot(g_ref[:N, :], w2_ref[...],
                           preferred_element_type=jnp.float32) + b2_ref[...]


def kernel(x, edge_index, W1, b1, W2, b2):
    out_ch = W2.shape[1]
    tc_f = pl.pallas_call(
        _tc_f_body,
        out_shape=jax.ShapeDtypeStruct((N, out_ch), jnp.float32),
    )
    src3 = edge_index[0].reshape(NW, NCH, CH)
    dst3 = edge_index[1].reshape(NW, NCH, CH)

    zeros32 = jnp.zeros((N_ACC, D_HID), jnp.float32)
    zeros_deg = jnp.zeros((N_ACC, D_DEG), jnp.float32)
    ones = jnp.ones((CH, D_DEG), jnp.float32)

    degp = _deg(dst3, ones, zeros_deg)
    h1 = _tc_b0(x, W1)
    u1, dinv = _b1(degp, h1)
    p1 = _prop(u1, src3, dst3, zeros32)
    u2 = _dml(p1, u1, dinv, b1.reshape(1, D_HID))
    p2 = _prop(u2, src3, dst3, zeros32)
    g = _g(p2, u2, dinv)
    out = tc_f(g, W2, b2.reshape(1, out_ch))
    return out
